# baseline (speedup 1.0000x reference)
import functools

import jax
import jax.numpy as jnp
from jax import lax
from jax.experimental import pallas as pl
from jax.experimental.pallas import tpu as pltpu
from jax.experimental.pallas import tpu_sc as plsc

V = 100000
H = 32
B = 1024
W = 20

NC = 2
NS = 16
NW = NC * NS
BPW = B // NW
IDX_PER_W = BPW * W
IDX_CHUNK = 128
N_GATHER = IDX_PER_W // IDX_CHUNK

VT = 2048


def _sc_pool(ctx_hbm, table_hbm, out_hbm, idx_v, rows_v, res_v, sem):
    wid = lax.axis_index("s") * NC + lax.axis_index("c")
    pltpu.sync_copy(ctx_hbm.at[pl.ds(wid * IDX_PER_W, IDX_PER_W)], idx_v)
    copies = [
        pltpu.async_copy(
            table_hbm.at[idx_v.at[pl.ds(j * IDX_CHUNK, IDX_CHUNK)]],
            rows_v.at[pl.ds(j * IDX_CHUNK, IDX_CHUNK)],
            sem,
        )
        for j in range(N_GATHER)
    ]
    for c in copies:
        c.wait()
    inv_w = jnp.float32(1.0 / W)
    for b in range(BPW):
        base = b * W
        for h in range(H // 16):
            acc = rows_v[base, pl.ds(h * 16, 16)]
            for w in range(1, W):
                acc = acc + rows_v[base + w, pl.ds(h * 16, 16)]
            res_v[b, pl.ds(h * 16, 16)] = acc * inv_w
    pltpu.sync_copy(res_v, out_hbm.at[pl.ds(wid * BPW, BPW)])


@functools.lru_cache(maxsize=1)
def _sc_pool_call():
    return functools.partial(
        pl.kernel,
        out_type=jax.ShapeDtypeStruct((B, H), jnp.float32),
        mesh=plsc.VectorSubcoreMesh(core_axis_name="c", subcore_axis_name="s"),
        scratch_types=[
            pltpu.VMEM((IDX_PER_W,), jnp.int32),
            pltpu.VMEM((IDX_PER_W, H), jnp.float32),
            pltpu.VMEM((BPW, H), jnp.float32),
            pltpu.SemaphoreType.DMA,
        ],
        compiler_params=pltpu.CompilerParams(use_tc_tiling_on_sc=False),
    )(_sc_pool)


def _mm_body(avg_ref, w_ref, b_ref, o_ref):
    o_ref[...] = (
        lax.dot_general(
            avg_ref[...],
            w_ref[...],
            dimension_numbers=(((1,), (1,)), ((), ())),
            preferred_element_type=jnp.float32,
        )
        + b_ref[...]
    )


def kernel(contexts, in_emb, out_W, out_b):
    ctx_flat = contexts.reshape(B * W).astype(jnp.int32)
    avg = _sc_pool_call()(ctx_flat, in_emb)
    y = pl.pallas_call(
        _mm_body,
        grid=(pl.cdiv(V, VT),),
        in_specs=[
            pl.BlockSpec((B, H), lambda i: (0, 0)),
            pl.BlockSpec((VT, H), lambda i: (i, 0)),
            pl.BlockSpec((1, VT), lambda i: (0, i)),
        ],
        out_specs=pl.BlockSpec((B, VT), lambda i: (0, i)),
        out_shape=jax.ShapeDtypeStruct((B, V), jnp.float32),
        compiler_params=pltpu.CompilerParams(
            dimension_semantics=("arbitrary",),
        ),
    )(avg, out_W, out_b.reshape(1, V))
    return y
